# Initial kernel scaffold; baseline (speedup 1.0000x reference)
import jax
import jax.numpy as jnp
from jax.experimental import pallas as pl


def kernel(attrs, edge_index, batch, W0_0, b0_0, W0_1, b0_1, W0_2, b0_2, W1_0, b1_0, W1_1, b1_1, W1_2, b1_2):
    raise NotImplementedError("write your pallas kernel here")


# speedup vs baseline: 4.1748x; 4.1748x over previous
import functools

import jax
import jax.numpy as jnp
from jax import lax
from jax.experimental import pallas as pl
from jax.experimental.pallas import tpu as pltpu
from jax.experimental.pallas import tpu_sc as plsc

NC = 2
NS = 16
NW = NC * NS
EW = 128

_HI = jax.lax.Precision.HIGHEST


def _make_sc_segment_sum(n, d, k, n_pad):
  mesh = plsc.VectorSubcoreMesh(core_axis_name="c", subcore_axis_name="s")
  rpt = (-(-n // NS) + 7) // 8 * 8
  rpt_last = n - (NS - 1) * rpt

  @functools.partial(
      pl.kernel,
      out_type=jax.ShapeDtypeStruct((2 * n, d), jnp.float32),
      mesh=mesh,
      scratch_types=[
          pltpu.VMEM((k, EW), jnp.int32),
          pltpu.VMEM((k, EW), jnp.int32),
          pltpu.VMEM((EW, d), jnp.float32),
          pltpu.VMEM_SHARED((n_pad, d), jnp.float32),
          pltpu.SemaphoreType.DMA,
      ],
  )
  def seg_sum(x_hbm, src_hbm, dst_hbm, out_hbm, src_v, dst_v, rows_v,
              acc_sh, sem):
    c = lax.axis_index("c")
    s = lax.axis_index("s")
    w = c * NS + s
    base = s * rpt

    @pl.when(s < NS - 1)
    def _():
      pltpu.sync_copy(x_hbm.at[pl.ds(base, rpt)],
                      acc_sh.at[pl.ds(base, rpt)])

    @pl.when(s == NS - 1)
    def _():
      pltpu.sync_copy(x_hbm.at[pl.ds(base, rpt_last)],
                      acc_sh.at[pl.ds(base, rpt_last)])

    pltpu.sync_copy(src_hbm.at[w], src_v)
    pltpu.sync_copy(dst_hbm.at[w], dst_v)
    plsc.subcore_barrier()

    @pl.loop(0, k)
    def _(j):
      pltpu.async_copy(x_hbm.at[src_v.at[j]], rows_v, sem).wait()
      pltpu.sync_copy(rows_v, acc_sh.at[dst_v.at[j]], add=True)

    plsc.subcore_barrier()

    @pl.when(s < NS - 1)
    def _():
      pltpu.sync_copy(acc_sh.at[pl.ds(base, rpt)],
                      out_hbm.at[pl.ds(c * n + base, rpt)])

    @pl.when(s == NS - 1)
    def _():
      pltpu.sync_copy(acc_sh.at[pl.ds(base, rpt_last)],
                      out_hbm.at[pl.ds(c * n + base, rpt_last)])

  return seg_sum


def _dot(a, b):
  return lax.dot_general(a, b, (((1,), (0,)), ((), ())),
                         precision=_HI, preferred_element_type=jnp.float32)


def _mlp_stack(u, w0, b0, w1, b1, w2, b2):
  h = jnp.tanh(_dot(u, w0) + b0)
  h = jnp.tanh(_dot(h, w1) + b1)
  h = jnp.tanh(_dot(h, w2) + b2)
  return jnp.tanh(h)


def _make_mlp(n, d, h, r):
  grid = n // r

  def body(p0_ref, p1_ref, x_ref, w0_ref, b0_ref, w1_ref, b1_ref, w2_ref,
           b2_ref, o_ref):
    u = p0_ref[...] + p1_ref[...] - x_ref[...]
    o_ref[...] = _mlp_stack(u, w0_ref[...], b0_ref[...], w1_ref[...],
                            b1_ref[...], w2_ref[...], b2_ref[...])

  row_spec = pl.BlockSpec((r, d), lambda i: (i, 0))
  return pl.pallas_call(
      body,
      grid=(grid,),
      in_specs=[
          row_spec, row_spec, row_spec,
          pl.BlockSpec((d, h), lambda i: (0, 0)),
          pl.BlockSpec((1, h), lambda i: (0, 0)),
          pl.BlockSpec((h, h), lambda i: (0, 0)),
          pl.BlockSpec((1, h), lambda i: (0, 0)),
          pl.BlockSpec((h, d), lambda i: (0, 0)),
          pl.BlockSpec((1, d), lambda i: (0, 0)),
      ],
      out_specs=row_spec,
      out_shape=jax.ShapeDtypeStruct((n, d), jnp.float32),
  )


def _make_mlp_readout(n, d, h, r, g):
  grid = n // r

  def body(p0_ref, p1_ref, x_ref, w0_ref, b0_ref, w1_ref, b1_ref, w2_ref,
           b2_ref, batch_ref, o_ref, sums_ref, counts_ref):
    i = pl.program_id(0)

    @pl.when(i == 0)
    def _():
      sums_ref[...] = jnp.zeros_like(sums_ref)
      counts_ref[...] = jnp.zeros_like(counts_ref)

    u = p0_ref[...] + p1_ref[...] - x_ref[...]
    xn = _mlp_stack(u, w0_ref[...], b0_ref[...], w1_ref[...], b1_ref[...],
                    w2_ref[...], b2_ref[...])
    gids = lax.broadcasted_iota(jnp.int32, (g, r), 0)
    onehot = (gids == batch_ref[0]).astype(jnp.float32)
    sums_ref[...] += lax.dot_general(
        onehot, xn, (((1,), (0,)), ((), ())),
        precision=_HI, preferred_element_type=jnp.float32)
    cnt = jnp.sum(onehot, axis=1, keepdims=True)
    counts_ref[...] += jnp.broadcast_to(cnt, (g, d))

    @pl.when(i == grid - 1)
    def _():
      o_ref[...] = sums_ref[...] / jnp.maximum(counts_ref[...], 1.0)

  row_spec = pl.BlockSpec((r, d), lambda i: (i, 0))
  return pl.pallas_call(
      body,
      grid=(grid,),
      in_specs=[
          row_spec, row_spec, row_spec,
          pl.BlockSpec((d, h), lambda i: (0, 0)),
          pl.BlockSpec((1, h), lambda i: (0, 0)),
          pl.BlockSpec((h, h), lambda i: (0, 0)),
          pl.BlockSpec((1, h), lambda i: (0, 0)),
          pl.BlockSpec((h, d), lambda i: (0, 0)),
          pl.BlockSpec((1, d), lambda i: (0, 0)),
          pl.BlockSpec((1, 1, r), lambda i: (i, 0, 0)),
      ],
      out_specs=pl.BlockSpec((g, d), lambda i: (0, 0)),
      out_shape=jax.ShapeDtypeStruct((g, d), jnp.float32),
      scratch_shapes=[
          pltpu.VMEM((g, d), jnp.float32),
          pltpu.VMEM((g, d), jnp.float32),
      ],
  )


def kernel(attrs, edge_index, batch,
           W0_0, b0_0, W0_1, b0_1, W0_2, b0_2,
           W1_0, b1_0, W1_1, b1_1, W1_2, b1_2):
  n, d = attrs.shape
  e = edge_index.shape[1]
  h = W0_0.shape[1]
  g = 64
  r = 1000
  k = -(-e // (NW * EW))
  e_pad = NW * k * EW
  n_pad = n + 16

  src = edge_index[0]
  dst = edge_index[1]
  pad = e_pad - e
  src3d = jnp.concatenate([src, jnp.zeros((pad,), jnp.int32)]).reshape(
      NW, k, EW)
  dst3d = jnp.concatenate([dst, jnp.full((pad,), n, jnp.int32)]).reshape(
      NW, k, EW)
  batch3d = batch.reshape(n // r, 1, r)
  b0s = (b0_0.reshape(1, h), b0_1.reshape(1, h), b0_2.reshape(1, d))
  b1s = (b1_0.reshape(1, h), b1_1.reshape(1, h), b1_2.reshape(1, d))

  seg_sum = _make_sc_segment_sum(n, d, k, n_pad)
  mlp = _make_mlp(n, d, h, r)
  mlp_readout = _make_mlp_readout(n, d, h, r, g)

  p = seg_sum(attrs, src3d, dst3d)
  x1 = mlp(p[:n], p[n:], attrs, W0_0, b0s[0], W0_1, b0s[1], W0_2, b0s[2])
  q = seg_sum(x1, src3d, dst3d)
  out = mlp_readout(q[:n], q[n:], x1, W1_0, b1s[0], W1_1, b1s[1],
                    W1_2, b1s[2], batch3d)
  return out
